# baseline (speedup 1.0000x reference)
import functools

import jax
import jax.numpy as jnp
from jax.experimental import pallas as pl
from jax.experimental.pallas import tpu as pltpu

B = 2048
K = 1024
D = 64
BLK = 128


def _split_bf16(x):
    hi = x.astype(jnp.bfloat16)
    lo = (x - hi.astype(jnp.float32)).astype(jnp.bfloat16)
    return hi, lo


def _mm(a, b):
    return jax.lax.dot_general(
        a, b, (((1,), (0,)), ((), ())),
        preferred_element_type=jnp.float32)


def _vq_block(ze_blk, emb_ref, ze_head_ref, out_ref):
    ze = ze_blk[...]
    emb = emb_ref[...]
    emb_sq = jnp.sum(emb * emb, axis=0, keepdims=True)
    ze_hi, ze_lo = _split_bf16(ze)
    emb_hi, emb_lo = _split_bf16(emb)
    dots = _mm(ze_hi, emb_hi) + (_mm(ze_hi, emb_lo) + _mm(ze_lo, emb_hi))
    scores = emb_sq - 2.0 * dots
    mins = jnp.min(scores, axis=1, keepdims=True)
    col = jax.lax.broadcasted_iota(jnp.int32, scores.shape, 1)
    idx = jnp.min(jnp.where(scores == mins, col, D), axis=1, keepdims=True)
    onehot = (col == idx).astype(jnp.bfloat16)
    zh_hi, zh_lo = _split_bf16(ze_head_ref[...])
    out_ref[...] = _mm(onehot, zh_hi) + _mm(onehot, zh_lo)


@functools.partial(jax.jit, static_argnames=())
def kernel(ze, emb):
    grid = (B // BLK,)
    return pl.pallas_call(
        _vq_block,
        grid=grid,
        in_specs=[
            pl.BlockSpec((BLK, K), lambda i: (i, 0)),
            pl.BlockSpec((K, D), lambda i: (0, 0)),
            pl.BlockSpec((D, K), lambda i: (0, 0)),
        ],
        out_specs=pl.BlockSpec((BLK, K), lambda i: (i, 0)),
        out_shape=jax.ShapeDtypeStruct((B, K), jnp.float32),
        compiler_params=pltpu.CompilerParams(
            dimension_semantics=("parallel",)),
    )(ze, emb, ze)


# speedup vs baseline: 1.2741x; 1.2741x over previous
import functools

import jax
import jax.numpy as jnp
from jax.experimental import pallas as pl
from jax.experimental.pallas import tpu as pltpu

B = 2048
K = 1024
D = 64
BLK = 512


def _split_bf16(x):
    hi = x.astype(jnp.bfloat16)
    lo = (x - hi.astype(jnp.float32)).astype(jnp.bfloat16)
    return hi, lo


def _mm(a, b):
    return jax.lax.dot_general(
        a, b, (((1,), (0,)), ((), ())),
        preferred_element_type=jnp.float32)


def _vq_block(ze_blk, emb_ref, ze_head_ref, out_ref):
    ze = ze_blk[...]
    emb = emb_ref[...]
    emb_sq = jnp.sum(emb * emb, axis=0, keepdims=True)
    ze_hi, ze_lo = _split_bf16(ze)
    emb_hi, emb_lo = _split_bf16(emb)
    dots = _mm(ze_hi, emb_hi) + (_mm(ze_hi, emb_lo) + _mm(ze_lo, emb_hi))
    scores = emb_sq - 2.0 * dots
    mins = jnp.min(scores, axis=1, keepdims=True)
    col = jax.lax.broadcasted_iota(jnp.int32, scores.shape, 1)
    idx = jnp.min(jnp.where(scores == mins, col, D), axis=1, keepdims=True)
    onehot = (col == idx).astype(jnp.bfloat16)
    zh_hi, zh_lo = _split_bf16(ze_head_ref[...])
    out_ref[...] = _mm(onehot, zh_hi) + _mm(onehot, zh_lo)


@functools.partial(jax.jit, static_argnames=())
def kernel(ze, emb):
    grid = (B // BLK,)
    return pl.pallas_call(
        _vq_block,
        grid=grid,
        in_specs=[
            pl.BlockSpec((BLK, K), lambda i: (i, 0)),
            pl.BlockSpec((K, D), lambda i: (0, 0)),
            pl.BlockSpec((D, K), lambda i: (0, 0)),
        ],
        out_specs=pl.BlockSpec((BLK, K), lambda i: (i, 0)),
        out_shape=jax.ShapeDtypeStruct((B, K), jnp.float32),
        compiler_params=pltpu.CompilerParams(
            dimension_semantics=("parallel",)),
    )(ze, emb, ze)


# speedup vs baseline: 1.4119x; 1.1082x over previous
import functools

import jax
import jax.numpy as jnp
from jax.experimental import pallas as pl
from jax.experimental.pallas import tpu as pltpu

B = 2048
K = 1024
D = 64
BLK = 256


def _split_bf16(x):
    hi = x.astype(jnp.bfloat16)
    lo = (x - hi.astype(jnp.float32)).astype(jnp.bfloat16)
    return hi, lo


def _mm(a, b):
    return jax.lax.dot_general(
        a, b, (((1,), (0,)), ((), ())),
        preferred_element_type=jnp.float32)


def _vq_block(ze_blk, emb_ref, ze_head_ref, out_ref, ehi_ref, elo_ref,
              esq_ref):
    @pl.when(pl.program_id(0) == 0)
    def _prep():
        emb = emb_ref[...]
        ehi, elo = _split_bf16(emb)
        ehi_ref[...] = ehi
        elo_ref[...] = elo
        esq_ref[...] = jnp.sum(emb * emb, axis=0, keepdims=True)

    ze = ze_blk[...]
    ze_hi, ze_lo = _split_bf16(ze)
    dots = (_mm(ze_hi, ehi_ref[...])
            + (_mm(ze_hi, elo_ref[...]) + _mm(ze_lo, ehi_ref[...])))
    scores = esq_ref[...] - 2.0 * dots
    mins = jnp.min(scores, axis=1, keepdims=True)
    col = jax.lax.broadcasted_iota(jnp.int32, scores.shape, 1)
    idx = jnp.min(jnp.where(scores == mins, col, D), axis=1, keepdims=True)
    onehot = (col == idx).astype(jnp.float32)
    out_ref[...] = _mm(onehot, ze_head_ref[...])


@functools.partial(jax.jit, static_argnames=())
def kernel(ze, emb):
    grid = (B // BLK,)
    return pl.pallas_call(
        _vq_block,
        grid=grid,
        in_specs=[
            pl.BlockSpec((BLK, K), lambda i: (i, 0)),
            pl.BlockSpec((K, D), lambda i: (0, 0)),
            pl.BlockSpec((D, K), lambda i: (0, 0)),
        ],
        out_specs=pl.BlockSpec((BLK, K), lambda i: (i, 0)),
        out_shape=jax.ShapeDtypeStruct((B, K), jnp.float32),
        scratch_shapes=[
            pltpu.VMEM((K, D), jnp.bfloat16),
            pltpu.VMEM((K, D), jnp.bfloat16),
            pltpu.VMEM((1, D), jnp.float32),
        ],
        compiler_params=pltpu.CompilerParams(
            dimension_semantics=("arbitrary",)),
    )(ze, emb, ze)
